# baseline (speedup 1.0000x reference)
import jax
import jax.numpy as jnp
from jax.experimental import pallas as pl


def _copy_kernel(x_ref, o_ref):
    o_ref[...] = x_ref[...]


def _sage(x, edge_index, edge_weight, W, b):
    src = edge_index[0]
    dst = edge_index[1]
    msg = x[src] * edge_weight[:, None]
    n = x.shape[0]
    s = jax.ops.segment_sum(msg, dst, num_segments=n)
    cnt = jax.ops.segment_sum(jnp.ones_like(edge_weight), dst, num_segments=n)
    mean = s / jnp.clip(cnt, 1.0)[:, None]
    h = jnp.concatenate([x, mean], axis=-1)
    return h @ W + b


def kernel(x, edge_index, edge_weight, W1, b1, W2, b2, W3, b3, Wl, bl):
    x1 = jax.nn.relu(_sage(x, edge_index, edge_weight, W1, b1))
    x2 = jax.nn.relu(_sage(x1, edge_index, edge_weight, W2, b2))
    x3 = jax.nn.relu(_sage(x2, edge_index, edge_weight, W3, b3))
    h = jnp.concatenate([x1, x2, x3], axis=-1)
    out = h @ Wl + bl
    out = pl.pallas_call(
        _copy_kernel,
        out_shape=jax.ShapeDtypeStruct(out.shape, out.dtype),
    )(out)
    return jax.nn.log_softmax(out, axis=-1)


# speedup vs baseline: 2.2477x; 2.2477x over previous
import functools

import jax
import jax.numpy as jnp
from jax import lax
from jax.experimental import pallas as pl
from jax.experimental.pallas import tpu as pltpu
from jax.experimental.pallas import tpu_sc as plsc

N = 10000
E = 320000
D = 128
H = 128
C = 7

NC = 2
NS = 16
NW = NC * NS
NPAD = 10240
K = 128
EPAD = 327680
EP = EPAD // NW
NCHUNK = EP // K
RPT = NPAD // NS


def _make_sage_sc():
    mesh = plsc.VectorSubcoreMesh(core_axis_name="c", subcore_axis_name="s")

    @functools.partial(
        pl.kernel,
        mesh=mesh,
        out_type=[
            jax.ShapeDtypeStruct((NC, NPAD, D), jnp.float32),
            jax.ShapeDtypeStruct((NC, NPAD), jnp.float32),
        ],
        scratch_types=[
            pltpu.VMEM((K,), jnp.int32),
            pltpu.VMEM((K,), jnp.int32),
            pltpu.VMEM((K,), jnp.float32),
            pltpu.VMEM((K,), jnp.float32),
            pltpu.VMEM((K, 16), jnp.float32),
            pltpu.VMEM((K, D), jnp.float32),
            pltpu.VMEM_SHARED((NPAD, D), jnp.float32),
            pltpu.VMEM_SHARED((NPAD,), jnp.float32),
            pltpu.SemaphoreType.DMA,
        ],
    )
    def sage_aggregate(x_hbm, src_hbm, dst_hbm, wgt_hbm,
                       out_hbm, cnt_out_hbm,
                       idx_v, dst_v, w_v, one_v, wx_v, rows_v,
                       acc_sh, cnt_sh, sem):
        cid = lax.axis_index("c")
        sid = lax.axis_index("s")
        wid = sid * NC + cid

        zero16 = jnp.zeros((16,), jnp.float32)
        one16 = jnp.ones((16,), jnp.float32)

        def zrow(i, carry):
            for f in range(D // 16):
                rows_v[i, pl.ds(f * 16, 16)] = zero16
            return carry
        lax.fori_loop(0, K, zrow, 0)
        for f in range(K // 16):
            w_v[pl.ds(f * 16, 16)] = zero16
            one_v[pl.ds(f * 16, 16)] = one16
        rbase = sid * RPT
        for j in range(RPT // K):
            pltpu.sync_copy(rows_v, acc_sh.at[pl.ds(rbase + j * K, K)])
            pltpu.sync_copy(w_v, cnt_sh.at[pl.ds(rbase + j * K, K)])
        plsc.subcore_barrier()

        def chunk(c, carry):
            ebase = wid * EP + c * K
            pltpu.sync_copy(src_hbm.at[pl.ds(ebase, K)], idx_v)
            pltpu.sync_copy(dst_hbm.at[pl.ds(ebase, K)], dst_v)
            pltpu.sync_copy(wgt_hbm.at[pl.ds(ebase, K)], wx_v)
            pltpu.async_copy(x_hbm.at[idx_v], rows_v, sem).wait()

            def mul_row(e, carry2):
                w = wx_v[e]
                for f in range(D // 16):
                    sl = pl.ds(f * 16, 16)
                    rows_v[e, sl] = rows_v[e, sl] * w
                return carry2
            lax.fori_loop(0, K, mul_row, 0)

            pltpu.sync_copy(rows_v, acc_sh.at[dst_v], add=True)
            pltpu.sync_copy(one_v, cnt_sh.at[dst_v], add=True)
            return carry
        lax.fori_loop(0, NCHUNK, chunk, 0)

        plsc.subcore_barrier()
        pltpu.sync_copy(acc_sh.at[pl.ds(rbase, RPT)],
                        out_hbm.at[cid, pl.ds(rbase, RPT)])
        pltpu.sync_copy(cnt_sh.at[pl.ds(rbase, RPT)],
                        cnt_out_hbm.at[cid, pl.ds(rbase, RPT)])

    return sage_aggregate


_sage_sc = _make_sage_sc()


def _tc_layer_body(x_ref, s0_ref, s1_ref, c0_ref, c1_ref,
                   wt_ref, wb_ref, b_ref, o_ref):
    cnt = c0_ref[...] + c1_ref[...]
    inv = 1.0 / jnp.maximum(cnt, 1.0)
    mean = (s0_ref[...] + s1_ref[...]) * inv
    h = (jnp.dot(x_ref[...], wt_ref[...], preferred_element_type=jnp.float32)
         + jnp.dot(mean, wb_ref[...], preferred_element_type=jnp.float32)
         + b_ref[...])
    o_ref[...] = jnp.maximum(h, 0.0)


def _tc_layer(x, s0, s1, c0, c1, W, b):
    B = 1024
    wt = W[:D]
    wb = W[D:]
    b2 = b.reshape(1, H)
    c0 = c0.reshape(NPAD, 1)
    c1 = c1.reshape(NPAD, 1)
    grid = NPAD // B
    return pl.pallas_call(
        _tc_layer_body,
        grid=(grid,),
        in_specs=[
            pl.BlockSpec((B, D), lambda i: (i, 0)),
            pl.BlockSpec((B, D), lambda i: (i, 0)),
            pl.BlockSpec((B, D), lambda i: (i, 0)),
            pl.BlockSpec((B, 1), lambda i: (i, 0)),
            pl.BlockSpec((B, 1), lambda i: (i, 0)),
            pl.BlockSpec((D, H), lambda i: (0, 0)),
            pl.BlockSpec((D, H), lambda i: (0, 0)),
            pl.BlockSpec((1, H), lambda i: (0, 0)),
        ],
        out_specs=pl.BlockSpec((B, H), lambda i: (i, 0)),
        out_shape=jax.ShapeDtypeStruct((NPAD, H), jnp.float32),
    )(x, s0, s1, c0, c1, wt, wb, b2)


def _tc_final_body(x1_ref, x2_ref, x3_ref, w1_ref, w2_ref, w3_ref,
                   b_ref, o_ref):
    z = (jnp.dot(x1_ref[...], w1_ref[...], preferred_element_type=jnp.float32)
         + jnp.dot(x2_ref[...], w2_ref[...], preferred_element_type=jnp.float32)
         + jnp.dot(x3_ref[...], w3_ref[...], preferred_element_type=jnp.float32)
         + b_ref[...])
    m = jnp.max(z, axis=-1, keepdims=True)
    e = jnp.exp(z - m)
    s = jnp.sum(e, axis=-1, keepdims=True)
    o_ref[...] = z - m - jnp.log(s)


def _tc_final(x1, x2, x3, Wl, bl):
    B = 1024
    CP = 128
    w_pad = jnp.zeros((3 * H, CP), jnp.float32).at[:, :C].set(Wl)
    b_pad = jnp.full((1, CP), -1e30, jnp.float32).at[0, :C].set(bl)
    grid = NPAD // B
    return pl.pallas_call(
        _tc_final_body,
        grid=(grid,),
        in_specs=[
            pl.BlockSpec((B, H), lambda i: (i, 0)),
            pl.BlockSpec((B, H), lambda i: (i, 0)),
            pl.BlockSpec((B, H), lambda i: (i, 0)),
            pl.BlockSpec((H, CP), lambda i: (0, 0)),
            pl.BlockSpec((H, CP), lambda i: (0, 0)),
            pl.BlockSpec((H, CP), lambda i: (0, 0)),
            pl.BlockSpec((1, CP), lambda i: (0, 0)),
        ],
        out_specs=pl.BlockSpec((B, CP), lambda i: (i, 0)),
        out_shape=jax.ShapeDtypeStruct((NPAD, CP), jnp.float32),
    )(x1, x2, x3, w_pad[:H], w_pad[H:2 * H], w_pad[2 * H:], b_pad)


def kernel(x, edge_index, edge_weight, W1, b1, W2, b2, W3, b3, Wl, bl):
    x = x.astype(jnp.float32)
    xp = jnp.zeros((NPAD, D), jnp.float32).at[:N].set(x)
    pad = EPAD - E
    src = jnp.pad(edge_index[0].astype(jnp.int32), (0, pad))
    dst = jnp.pad(edge_index[1].astype(jnp.int32), (0, pad),
                  constant_values=NPAD - 1)
    wgt = jnp.pad(edge_weight.astype(jnp.float32), (0, pad))
    w_exp = jnp.repeat(wgt[:, None], 16, axis=1)

    def layer(xin, W, b):
        sums, cnts = _sage_sc(xin, src, dst, w_exp)
        return _tc_layer(xin, sums[0], sums[1], cnts[0], cnts[1], W, b)

    x1 = layer(xp, W1, b1)
    x2 = layer(x1, W2, b2)
    x3 = layer(x2, W3, b3)
    out = _tc_final(x1, x2, x3, Wl, bl)
    return out[:N, :C]


# speedup vs baseline: 3.8312x; 1.7045x over previous
import functools

import jax
import jax.numpy as jnp
from jax import lax
from jax.experimental import pallas as pl
from jax.experimental.pallas import tpu as pltpu
from jax.experimental.pallas import tpu_sc as plsc

N = 10000
E = 320000
D = 128
H = 128
C = 7

NC = 2
NS = 16
DG = D // NC
NPAD = 10240
K = 128
EPAD = 327680
EP = EPAD // NS
NCHUNK = EP // K
RPT = NPAD // NS

NROWBUF = 2
NMETA = 4


def _make_sage_sc():
    mesh = plsc.VectorSubcoreMesh(core_axis_name="c", subcore_axis_name="s")

    @functools.partial(
        pl.kernel,
        mesh=mesh,
        compiler_params=pltpu.CompilerParams(use_tc_tiling_on_sc=False),
        out_type=[
            jax.ShapeDtypeStruct((NC, NPAD, DG), jnp.float32),
            jax.ShapeDtypeStruct((NC, NPAD), jnp.float32),
        ],
        scratch_types=(
            [
                pltpu.VMEM((K,), jnp.float32),
                pltpu.VMEM((K,), jnp.float32),
            ]
            + [pltpu.VMEM((K,), jnp.int32)] * NMETA
            + [pltpu.VMEM((K,), jnp.int32)] * NMETA
            + [pltpu.VMEM((K, 16), jnp.float32)] * NMETA
            + [pltpu.VMEM((K, DG), jnp.float32)] * NROWBUF
            + [
                pltpu.VMEM_SHARED((NPAD, DG), jnp.float32),
                pltpu.VMEM_SHARED((NPAD,), jnp.float32),
            ]
            + [pltpu.SemaphoreType.DMA] * (NMETA + 2 * NROWBUF)
        ),
    )
    def sage_aggregate(x_hbm, src_hbm, dst_hbm, wgt_hbm,
                       out_hbm, cnt_out_hbm,
                       z_v, one_v, *rest):
        idx = rest[:NMETA]
        dsts = rest[NMETA:2 * NMETA]
        wx = rest[2 * NMETA:3 * NMETA]
        rows = rest[3 * NMETA:3 * NMETA + NROWBUF]
        acc_sh = rest[3 * NMETA + NROWBUF]
        cnt_sh = rest[3 * NMETA + NROWBUF + 1]
        sems = rest[3 * NMETA + NROWBUF + 2:]
        sm = sems[:NMETA]
        sg = sems[NMETA:NMETA + NROWBUF]
        ss = sems[NMETA + NROWBUF:]

        cid = lax.axis_index("c")
        sid = lax.axis_index("s")
        cbase = sid * NCHUNK
        roff = cid * NPAD

        zero16 = jnp.zeros((16,), jnp.float32)
        one16 = jnp.ones((16,), jnp.float32)

        def zrow(i, carry):
            for f in range(DG // 16):
                rows[0][i, pl.ds(f * 16, 16)] = zero16
            return carry
        lax.fori_loop(0, K, zrow, 0)
        for f in range(K // 16):
            z_v[pl.ds(f * 16, 16)] = zero16
            one_v[pl.ds(f * 16, 16)] = one16
        rbase = sid * RPT
        for j in range(RPT // K):
            pltpu.async_copy(rows[0], acc_sh.at[pl.ds(rbase + j * K, K)],
                             sg[0])
            pltpu.async_copy(z_v, cnt_sh.at[pl.ds(rbase + j * K, K)], sg[1])
        for j in range(RPT // K):
            pltpu.make_async_copy(rows[0], acc_sh.at[pl.ds(rbase, K)],
                                  sg[0]).wait()
            pltpu.make_async_copy(z_v, cnt_sh.at[pl.ds(rbase, K)],
                                  sg[1]).wait()
        plsc.subcore_barrier()

        def meta_fetch(c, q):
            pltpu.async_copy(src_hbm.at[cbase + c], idx[q], sm[q])
            pltpu.async_copy(dst_hbm.at[cbase + c], dsts[q], sm[q])
            pltpu.async_copy(wgt_hbm.at[cbase + c], wx[q], sm[q])

        def meta_wait(q):
            pltpu.make_async_copy(src_hbm.at[cbase], idx[q], sm[q]).wait()
            pltpu.make_async_copy(dst_hbm.at[cbase], dsts[q], sm[q]).wait()
            pltpu.make_async_copy(wgt_hbm.at[cbase], wx[q], sm[q]).wait()
            for f in range(K // 16):
                sl = pl.ds(f * 16, 16)
                idx[q][sl] = idx[q][sl] + roff

        meta_fetch(0, 0)
        meta_fetch(1, 1)
        meta_wait(0)
        pltpu.async_copy(x_hbm.at[idx[0]], rows[0], sg[0])

        def chunk_step(c, b, q, qn, bn):
            @pl.when(c + 2 < NCHUNK)
            def _():
                meta_fetch(c + 2, (q + 2) % NMETA)

            @pl.when(c >= 1)
            def _():
                pltpu.make_async_copy(rows[bn], acc_sh.at[dsts[0]],
                                      ss[bn]).wait()
                pltpu.make_async_copy(one_v, cnt_sh.at[dsts[0]],
                                      ss[bn]).wait()

            @pl.when(c + 1 < NCHUNK)
            def _():
                meta_wait(qn)
                pltpu.async_copy(x_hbm.at[idx[qn]], rows[bn], sg[bn])

            pltpu.make_async_copy(x_hbm.at[idx[0]], rows[b], sg[b]).wait()

            def mul_row(e, carry2):
                w = wx[q][e]
                for f in range(DG // 16):
                    sl = pl.ds(f * 16, 16)
                    rows[b][e, sl] = rows[b][e, sl] * w
                return carry2
            lax.fori_loop(0, K, mul_row, 0, unroll=2)

            pltpu.async_copy(rows[b], acc_sh.at[dsts[q]], ss[b], add=True)
            pltpu.async_copy(one_v, cnt_sh.at[dsts[q]], ss[b], add=True)

        def round_body(r, carry):
            g = r * NMETA
            for j in range(NMETA):
                c = g + j
                chunk_step(c, j % NROWBUF, j, (j + 1) % NMETA,
                           (j + 1) % NROWBUF)
            return carry
        lax.fori_loop(0, NCHUNK // NMETA, round_body, 0)

        bl_ = (NCHUNK - 1) % NROWBUF
        pltpu.make_async_copy(rows[bl_], acc_sh.at[dsts[0]], ss[bl_]).wait()
        pltpu.make_async_copy(one_v, cnt_sh.at[dsts[0]], ss[bl_]).wait()

        plsc.subcore_barrier()
        pltpu.async_copy(acc_sh.at[pl.ds(rbase, RPT)],
                         out_hbm.at[cid, pl.ds(rbase, RPT)], sg[0])
        pltpu.async_copy(cnt_sh.at[pl.ds(rbase, RPT)],
                         cnt_out_hbm.at[cid, pl.ds(rbase, RPT)], sg[1])
        pltpu.make_async_copy(acc_sh.at[pl.ds(rbase, RPT)],
                              out_hbm.at[cid, pl.ds(rbase, RPT)],
                              sg[0]).wait()
        pltpu.make_async_copy(cnt_sh.at[pl.ds(rbase, RPT)],
                              cnt_out_hbm.at[cid, pl.ds(rbase, RPT)],
                              sg[1]).wait()

    return sage_aggregate


_sage_sc = _make_sage_sc()


def _tc_layer_body(xlo_ref, xhi_ref, slo_ref, shi_ref, c0_ref,
                   wtl_ref, wth_ref, wbl_ref, wbh_ref, b_ref, o_ref):
    inv = 1.0 / jnp.maximum(c0_ref[...], 1.0)
    dot = functools.partial(jnp.dot, preferred_element_type=jnp.float32)
    h = (dot(xlo_ref[0], wtl_ref[...])
         + dot(xhi_ref[0], wth_ref[...])
         + dot(slo_ref[0] * inv, wbl_ref[...])
         + dot(shi_ref[0] * inv, wbh_ref[...])
         + b_ref[...])
    h = jnp.maximum(h, 0.0)
    o_ref[0] = h[:, :DG]
    o_ref[1] = h[:, DG:]


def _tc_layer(x_split, slo, shi, c0, W, b):
    B = 1024
    b2 = b.reshape(1, H)
    c0 = c0.reshape(NPAD, 1)
    grid = NPAD // B
    x3 = x_split.reshape(NC, NPAD, DG)
    s3lo = slo.reshape(1, NPAD, DG)
    s3hi = shi.reshape(1, NPAD, DG)
    out = pl.pallas_call(
        _tc_layer_body,
        grid=(grid,),
        in_specs=[
            pl.BlockSpec((1, B, DG), lambda i: (0, i, 0)),
            pl.BlockSpec((1, B, DG), lambda i: (1, i, 0)),
            pl.BlockSpec((1, B, DG), lambda i: (0, i, 0)),
            pl.BlockSpec((1, B, DG), lambda i: (0, i, 0)),
            pl.BlockSpec((B, 1), lambda i: (i, 0)),
            pl.BlockSpec((DG, H), lambda i: (0, 0)),
            pl.BlockSpec((DG, H), lambda i: (0, 0)),
            pl.BlockSpec((DG, H), lambda i: (0, 0)),
            pl.BlockSpec((DG, H), lambda i: (0, 0)),
            pl.BlockSpec((1, H), lambda i: (0, 0)),
        ],
        out_specs=pl.BlockSpec((NC, B, DG), lambda i: (0, i, 0)),
        out_shape=jax.ShapeDtypeStruct((NC, NPAD, DG), jnp.float32),
    )(x3, x3, s3lo, s3hi, c0, W[:DG], W[DG:D], W[D:D + DG], W[D + DG:], b2)
    return out.reshape(NC * NPAD, DG)


def _tc_final_body(x1l, x1h, x2l, x2h, x3l, x3h,
                   w1l, w1h, w2l, w2h, w3l, w3h, b_ref, o_ref):
    dot = functools.partial(jnp.dot, preferred_element_type=jnp.float32)
    z = (dot(x1l[0], w1l[...]) + dot(x1h[0], w1h[...])
         + dot(x2l[0], w2l[...]) + dot(x2h[0], w2h[...])
         + dot(x3l[0], w3l[...]) + dot(x3h[0], w3h[...])
         + b_ref[...])
    m = jnp.max(z, axis=-1, keepdims=True)
    e = jnp.exp(z - m)
    s = jnp.sum(e, axis=-1, keepdims=True)
    o_ref[...] = z - m - jnp.log(s)


def _tc_final(x1, x2, x3, Wl, bl):
    B = 1024
    CP = 128
    w_pad = jnp.zeros((3 * H, CP), jnp.float32).at[:, :C].set(Wl)
    b_pad = jnp.full((1, CP), -1e30, jnp.float32).at[0, :C].set(bl)
    grid = NPAD // B
    xs = [x.reshape(NC, NPAD, DG) for x in (x1, x2, x3)]
    lo = lambda i: (0, i, 0)
    hi = lambda i: (1, i, 0)
    ws = [w_pad[j * DG:(j + 1) * DG] for j in range(6)]
    return pl.pallas_call(
        _tc_final_body,
        grid=(grid,),
        in_specs=(
            [pl.BlockSpec((1, B, DG), m) for m in (lo, hi, lo, hi, lo, hi)]
            + [pl.BlockSpec((DG, CP), lambda i: (0, 0))] * 6
            + [pl.BlockSpec((1, CP), lambda i: (0, 0))]
        ),
        out_specs=pl.BlockSpec((B, CP), lambda i: (i, 0)),
        out_shape=jax.ShapeDtypeStruct((NPAD, CP), jnp.float32),
    )(xs[0], xs[0], xs[1], xs[1], xs[2], xs[2], *ws, b_pad)


def kernel(x, edge_index, edge_weight, W1, b1, W2, b2, W3, b3, Wl, bl):
    x = x.astype(jnp.float32)
    xp = jnp.zeros((NPAD, D), jnp.float32).at[:N].set(x)
    x_split = jnp.concatenate([xp[:, :DG], xp[:, DG:]], axis=0)
    pad = EPAD - E
    src = jnp.pad(edge_index[0].astype(jnp.int32), (0, pad))
    dst = jnp.pad(edge_index[1].astype(jnp.int32), (0, pad),
                  constant_values=NPAD - 1)
    wgt = jnp.pad(edge_weight.astype(jnp.float32), (0, pad))
    srcm = src.reshape(EPAD // K, K)
    dstm = dst.reshape(EPAD // K, K)
    wxm = jnp.repeat(wgt[:, None], 16, axis=1).reshape(EPAD // K, K, 16)

    def layer(xin_split, W, b):
        sums, cnts = _sage_sc(xin_split, srcm, dstm, wxm)
        return _tc_layer(xin_split, sums[0], sums[1], cnts[0], W, b)

    x1 = layer(x_split, W1, b1)
    x2 = layer(x1, W2, b2)
    x3 = layer(x2, W3, b3)
    out = _tc_final(x1, x2, x3, Wl, bl)
    return out[:N, :C]


# speedup vs baseline: 6.0297x; 1.5738x over previous
import functools

import jax
import jax.numpy as jnp
from jax import lax
from jax.experimental import pallas as pl
from jax.experimental.pallas import tpu as pltpu
from jax.experimental.pallas import tpu_sc as plsc

N = 10000
E = 320000
D = 128
H = 128
C = 7

NC = 2
NS = 16
DG = D // NC
NPAD = 10240
K = 128
EPAD = 327680
EP = EPAD // NS
NCHUNK = EP // K
RPT = NPAD // NS

NROWBUF = 2
NMETA = 4


def _make_sage_sc():
    mesh = plsc.VectorSubcoreMesh(core_axis_name="c", subcore_axis_name="s")

    @functools.partial(
        pl.kernel,
        mesh=mesh,
        compiler_params=pltpu.CompilerParams(use_tc_tiling_on_sc=False, needs_layout_passes=False),
        out_type=[
            jax.ShapeDtypeStruct((NC, NPAD, DG), jnp.float32),
            jax.ShapeDtypeStruct((NC, NPAD), jnp.float32),
        ],
        scratch_types=(
            [
                pltpu.VMEM((K,), jnp.float32),
                pltpu.VMEM((K,), jnp.float32),
            ]
            + [pltpu.VMEM((K,), jnp.int32)] * NMETA
            + [pltpu.VMEM((K,), jnp.int32)] * NMETA
            + [pltpu.VMEM((K,), jnp.float32)] * NMETA
            + [pltpu.VMEM((K, DG), jnp.float32)] * NROWBUF
            + [
                pltpu.VMEM_SHARED((NPAD, DG), jnp.float32),
                pltpu.VMEM_SHARED((NPAD, DG), jnp.float32),
                pltpu.VMEM_SHARED((NPAD,), jnp.float32),
            ]
            + [pltpu.SemaphoreType.DMA] * (NMETA + 2 * NROWBUF)
        ),
    )
    def sage_aggregate(x_hbm, src_hbm, dst_hbm, wgt_hbm,
                       out_hbm, cnt_out_hbm,
                       z_v, one_v, *rest):
        idx = rest[:NMETA]
        dsts = rest[NMETA:2 * NMETA]
        wx = rest[2 * NMETA:3 * NMETA]
        rows = rest[3 * NMETA:3 * NMETA + NROWBUF]
        acc_sh = rest[3 * NMETA + NROWBUF]
        x_sh = rest[3 * NMETA + NROWBUF + 1]
        cnt_sh = rest[3 * NMETA + NROWBUF + 2]
        sems = rest[3 * NMETA + NROWBUF + 3:]
        sm = sems[:NMETA]
        sg = sems[NMETA:NMETA + NROWBUF]
        ss = sems[NMETA + NROWBUF:]

        cid = lax.axis_index("c")
        sid = lax.axis_index("s")
        cbase = sid * NCHUNK
        roff = cid * NPAD

        zero16 = jnp.zeros((16,), jnp.float32)
        one16 = jnp.ones((16,), jnp.float32)

        def zrow(i, carry):
            for f in range(DG // 16):
                rows[0][i, pl.ds(f * 16, 16)] = zero16
            return carry
        lax.fori_loop(0, K, zrow, 0)
        for f in range(K // 16):
            z_v[pl.ds(f * 16, 16)] = zero16
            one_v[pl.ds(f * 16, 16)] = one16
        rbase = sid * RPT
        pltpu.async_copy(x_hbm.at[pl.ds(roff + rbase, RPT)],
                         x_sh.at[pl.ds(rbase, RPT)], sg[1])
        for j in range(RPT // K):
            pltpu.async_copy(rows[0], acc_sh.at[pl.ds(rbase + j * K, K)],
                             sg[0])
        pltpu.async_copy(z_v, cnt_sh.at[pl.ds(rbase, K)], sg[0])
        for j in range(RPT // K - 1):
            pltpu.async_copy(z_v, cnt_sh.at[pl.ds(rbase + (j + 1) * K, K)],
                             sg[0])
        for j in range(RPT // K):
            pltpu.make_async_copy(rows[0], acc_sh.at[pl.ds(rbase, K)],
                                  sg[0]).wait()
            pltpu.make_async_copy(z_v, cnt_sh.at[pl.ds(rbase, K)],
                                  sg[0]).wait()
        pltpu.make_async_copy(x_hbm.at[pl.ds(roff + rbase, RPT)],
                              x_sh.at[pl.ds(rbase, RPT)], sg[1]).wait()
        plsc.subcore_barrier()

        def meta_fetch(c, q):
            pltpu.async_copy(src_hbm.at[cbase + c], idx[q], sm[q])
            pltpu.async_copy(dst_hbm.at[cbase + c], dsts[q], sm[q])
            pltpu.async_copy(wgt_hbm.at[cbase + c], wx[q], sm[q])

        def meta_wait(q):
            pltpu.make_async_copy(src_hbm.at[cbase], idx[q], sm[q]).wait()
            pltpu.make_async_copy(dst_hbm.at[cbase], dsts[q], sm[q]).wait()
            pltpu.make_async_copy(wgt_hbm.at[cbase], wx[q], sm[q]).wait()

        meta_fetch(0, 0)
        meta_fetch(1, 1)
        meta_wait(0)
        pltpu.async_copy(x_sh.at[idx[0]], rows[0], sg[0])

        def chunk_step(c, b, q, qn, bn):
            @pl.when(c + 2 < NCHUNK)
            def _():
                meta_fetch(c + 2, (q + 2) % NMETA)

            @pl.when(c >= 1)
            def _():
                pltpu.make_async_copy(rows[bn], acc_sh.at[dsts[0]],
                                      ss[bn]).wait()
                pltpu.make_async_copy(one_v, cnt_sh.at[dsts[0]],
                                      ss[bn]).wait()

            @pl.when(c + 1 < NCHUNK)
            def _():
                meta_wait(qn)
                pltpu.async_copy(x_sh.at[idx[qn]], rows[bn], sg[bn])

            pltpu.make_async_copy(x_sh.at[idx[0]], rows[b], sg[b]).wait()

            def mul_row(e, carry2):
                w = plsc.load_gather(wx[q], [jnp.full((16,), e, jnp.int32)])
                for f in range(DG // 16):
                    sl = pl.ds(f * 16, 16)
                    rows[b][e, sl] = rows[b][e, sl] * w
                return carry2
            lax.fori_loop(0, K, mul_row, 0, unroll=2)

            pltpu.async_copy(rows[b], acc_sh.at[dsts[q]], ss[b], add=True)
            pltpu.async_copy(one_v, cnt_sh.at[dsts[q]], ss[b], add=True)

        def round_body(r, carry):
            g = r * NMETA
            for j in range(NMETA):
                c = g + j
                chunk_step(c, j % NROWBUF, j, (j + 1) % NMETA,
                           (j + 1) % NROWBUF)
            return carry
        lax.fori_loop(0, NCHUNK // NMETA, round_body, 0)

        bl_ = (NCHUNK - 1) % NROWBUF
        pltpu.make_async_copy(rows[bl_], acc_sh.at[dsts[0]], ss[bl_]).wait()
        pltpu.make_async_copy(one_v, cnt_sh.at[dsts[0]], ss[bl_]).wait()

        plsc.subcore_barrier()
        pltpu.async_copy(acc_sh.at[pl.ds(rbase, RPT)],
                         out_hbm.at[cid, pl.ds(rbase, RPT)], sg[0])
        pltpu.async_copy(cnt_sh.at[pl.ds(rbase, RPT)],
                         cnt_out_hbm.at[cid, pl.ds(rbase, RPT)], sg[1])
        pltpu.make_async_copy(acc_sh.at[pl.ds(rbase, RPT)],
                              out_hbm.at[cid, pl.ds(rbase, RPT)],
                              sg[0]).wait()
        pltpu.make_async_copy(cnt_sh.at[pl.ds(rbase, RPT)],
                              cnt_out_hbm.at[cid, pl.ds(rbase, RPT)],
                              sg[1]).wait()

    return sage_aggregate


_sage_sc = _make_sage_sc()


def _tc_layer_body(xlo_ref, xhi_ref, slo_ref, shi_ref, c0_ref,
                   wtl_ref, wth_ref, wbl_ref, wbh_ref, b_ref, o_ref):
    inv = 1.0 / jnp.maximum(c0_ref[...], 1.0)
    dot = functools.partial(jnp.dot, preferred_element_type=jnp.float32)
    h = (dot(xlo_ref[0], wtl_ref[...])
         + dot(xhi_ref[0], wth_ref[...])
         + dot(slo_ref[0] * inv, wbl_ref[...])
         + dot(shi_ref[0] * inv, wbh_ref[...])
         + b_ref[...])
    h = jnp.maximum(h, 0.0)
    o_ref[0] = h[:, :DG]
    o_ref[1] = h[:, DG:]


def _tc_layer(x_split, slo, shi, c0, W, b):
    B = 1024
    b2 = b.reshape(1, H)
    c0 = c0.reshape(NPAD, 1)
    grid = NPAD // B
    x3 = x_split.reshape(NC, NPAD, DG)
    s3lo = slo.reshape(1, NPAD, DG)
    s3hi = shi.reshape(1, NPAD, DG)
    out = pl.pallas_call(
        _tc_layer_body,
        grid=(grid,),
        in_specs=[
            pl.BlockSpec((1, B, DG), lambda i: (0, i, 0)),
            pl.BlockSpec((1, B, DG), lambda i: (1, i, 0)),
            pl.BlockSpec((1, B, DG), lambda i: (0, i, 0)),
            pl.BlockSpec((1, B, DG), lambda i: (0, i, 0)),
            pl.BlockSpec((B, 1), lambda i: (i, 0)),
            pl.BlockSpec((DG, H), lambda i: (0, 0)),
            pl.BlockSpec((DG, H), lambda i: (0, 0)),
            pl.BlockSpec((DG, H), lambda i: (0, 0)),
            pl.BlockSpec((DG, H), lambda i: (0, 0)),
            pl.BlockSpec((1, H), lambda i: (0, 0)),
        ],
        out_specs=pl.BlockSpec((NC, B, DG), lambda i: (0, i, 0)),
        out_shape=jax.ShapeDtypeStruct((NC, NPAD, DG), jnp.float32),
    )(x3, x3, s3lo, s3hi, c0, W[:DG], W[DG:D], W[D:D + DG], W[D + DG:], b2)
    return out.reshape(NC * NPAD, DG)


def _tc_final_body(x1l, x1h, x2l, x2h, x3l, x3h,
                   w1l, w1h, w2l, w2h, w3l, w3h, b_ref, o_ref):
    dot = functools.partial(jnp.dot, preferred_element_type=jnp.float32)
    z = (dot(x1l[0], w1l[...]) + dot(x1h[0], w1h[...])
         + dot(x2l[0], w2l[...]) + dot(x2h[0], w2h[...])
         + dot(x3l[0], w3l[...]) + dot(x3h[0], w3h[...])
         + b_ref[...])
    m = jnp.max(z, axis=-1, keepdims=True)
    e = jnp.exp(z - m)
    s = jnp.sum(e, axis=-1, keepdims=True)
    o_ref[...] = z - m - jnp.log(s)


def _tc_final(x1, x2, x3, Wl, bl):
    B = 1024
    CP = 128
    w_pad = jnp.zeros((3 * H, CP), jnp.float32).at[:, :C].set(Wl)
    b_pad = jnp.full((1, CP), -1e30, jnp.float32).at[0, :C].set(bl)
    grid = NPAD // B
    xs = [x.reshape(NC, NPAD, DG) for x in (x1, x2, x3)]
    lo = lambda i: (0, i, 0)
    hi = lambda i: (1, i, 0)
    ws = [w_pad[j * DG:(j + 1) * DG] for j in range(6)]
    return pl.pallas_call(
        _tc_final_body,
        grid=(grid,),
        in_specs=(
            [pl.BlockSpec((1, B, DG), m) for m in (lo, hi, lo, hi, lo, hi)]
            + [pl.BlockSpec((DG, CP), lambda i: (0, 0))] * 6
            + [pl.BlockSpec((1, CP), lambda i: (0, 0))]
        ),
        out_specs=pl.BlockSpec((B, CP), lambda i: (i, 0)),
        out_shape=jax.ShapeDtypeStruct((NPAD, CP), jnp.float32),
    )(xs[0], xs[0], xs[1], xs[1], xs[2], xs[2], *ws, b_pad)


def kernel(x, edge_index, edge_weight, W1, b1, W2, b2, W3, b3, Wl, bl):
    x = x.astype(jnp.float32)
    xp = jnp.zeros((NPAD, D), jnp.float32).at[:N].set(x)
    x_split = jnp.concatenate([xp[:, :DG], xp[:, DG:]], axis=0)
    pad = EPAD - E
    src = jnp.pad(edge_index[0].astype(jnp.int32), (0, pad))
    dst = jnp.pad(edge_index[1].astype(jnp.int32), (0, pad),
                  constant_values=NPAD - 1)
    wgt = jnp.pad(edge_weight.astype(jnp.float32), (0, pad))
    srcm = src.reshape(EPAD // K, K)
    dstm = dst.reshape(EPAD // K, K)
    wxm = wgt.reshape(EPAD // K, K)

    def layer(xin_split, W, b):
        sums, cnts = _sage_sc(xin_split, srcm, dstm, wxm)
        return _tc_layer(xin_split, sums[0], sums[1], cnts[0], W, b)

    x1 = layer(x_split, W1, b1)
    x2 = layer(x1, W2, b2)
    x3 = layer(x2, W3, b3)
    out = _tc_final(x1, x2, x3, Wl, bl)
    return out[:N, :C]
